# baseline (speedup 1.0000x reference)
import functools

import jax
import jax.numpy as jnp
from jax import lax
from jax.experimental import pallas as pl
from jax.experimental.pallas import tpu as pltpu
from jax.experimental.pallas import tpu_sc as plsc

_CH = 16


@functools.lru_cache(maxsize=None)
def _build_gather(B, V, D):
    info = plsc.get_sparse_core_info()
    NC, NS, L = info.num_cores, info.num_subcores, info.num_lanes
    NW = NC * NS
    assert B % (NW * _CH) == 0 and V % 8 == 0 and D == L, (B, V, D)
    b_per_w = B // NW
    n_chunks = b_per_w // _CH

    mesh = plsc.VectorSubcoreMesh(core_axis_name="c", subcore_axis_name="s")

    @functools.partial(
        pl.kernel,
        mesh=mesh,
        out_type=jax.ShapeDtypeStruct((B, D), jnp.float32),
        scratch_types=[
            pltpu.VMEM((b_per_w,), jnp.int32),
            pltpu.VMEM((2 * _CH * 8, D), jnp.float32),
            pltpu.VMEM((b_per_w, D), jnp.float32),
            pltpu.SemaphoreType.DMA,
            pltpu.SemaphoreType.DMA,
        ],
        compiler_params=pltpu.CompilerParams(
            use_tc_tiling_on_sc=True, needs_layout_passes=False
        ),
    )
    def gather_kernel(ids_hbm, table_hbm, out_hbm, idx_v, slots_v, rows_v,
                      sem0, sem1):
        wid = lax.axis_index("s") * NC + lax.axis_index("c")
        base = wid * b_per_w
        pltpu.sync_copy(ids_hbm.at[pl.ds(base, b_per_w)], idx_v)

        lanes = lax.iota(jnp.int32, L)

        def fire(m, sem):
            off = (m % 2) * _CH
            tiles = lax.shift_right_logical(idx_v[pl.ds(m * _CH, _CH)], 3)
            for j in range(_CH):
                t = jnp.sum(jnp.where(lanes == j, tiles, 0))
                pltpu.async_copy(
                    table_hbm.at[pl.ds(t * 8, 8), :],
                    slots_v.at[pl.ds((off + j) * 8, 8), :],
                    sem,
                )

        def drain(sem):
            for _ in range(_CH):
                pltpu.make_async_copy(
                    table_hbm.at[pl.ds(0, 8), :],
                    slots_v.at[pl.ds(0, 8), :],
                    sem,
                ).wait()

        def extract(m):
            off = (m % 2) * _CH
            s_vec = idx_v[pl.ds(m * _CH, _CH)] & 7
            slot_vec = (off + lanes) * 8 + s_vec
            k_vec = m * _CH + lanes
            for j in range(D):
                j_vec = jnp.full((L,), j, jnp.int32)
                row_j = plsc.load_gather(slots_v, [slot_vec, j_vec])
                plsc.store_scatter(rows_v, [k_vec, j_vec], row_j)

        def body(m, carry):
            @pl.when(m % 2 == 0)
            def _():
                fire(m, sem0)

            @pl.when(m % 2 == 1)
            def _():
                fire(m, sem1)

            @pl.when(m % 2 == 1)
            def _():
                drain(sem0)
                extract(m - 1)

            @pl.when((m % 2 == 0) & (m > 0))
            def _():
                drain(sem1)
                extract(m - 1)

            return carry

        lax.fori_loop(0, n_chunks, body, 0)
        drain(sem1 if (n_chunks % 2 == 0) else sem0)
        extract(n_chunks - 1)

        pltpu.sync_copy(rows_v, out_hbm.at[pl.ds(base, b_per_w), :])

    return gather_kernel


def kernel(speaker_ids, table):
    ids = speaker_ids.astype(jnp.int32)
    (B,) = ids.shape
    V, D = table.shape
    return _build_gather(B, V, D)(ids, table)


# speedup vs baseline: 1.6226x; 1.6226x over previous
import functools

import jax
import jax.numpy as jnp
from jax import lax
from jax.experimental import pallas as pl
from jax.experimental.pallas import tpu as pltpu
from jax.experimental.pallas import tpu_sc as plsc

_CH = 16


@functools.lru_cache(maxsize=None)
def _build_gather(B, V, D):
    info = plsc.get_sparse_core_info()
    NC, NS, L = info.num_cores, info.num_subcores, info.num_lanes
    NW = NC * NS
    assert B % (NW * _CH) == 0 and V % 8 == 0 and D == L, (B, V, D)
    b_per_w = B // NW
    n_chunks = b_per_w // _CH

    mesh = plsc.VectorSubcoreMesh(core_axis_name="c", subcore_axis_name="s")

    @functools.partial(
        pl.kernel,
        mesh=mesh,
        out_type=jax.ShapeDtypeStruct((B, D), jnp.float32),
        scratch_types=[
            pltpu.VMEM((b_per_w,), jnp.int32),
            pltpu.VMEM((2 * _CH, 8, D), jnp.float32),
            pltpu.VMEM((b_per_w, D), jnp.float32),
            pltpu.SemaphoreType.DMA,
            pltpu.SemaphoreType.DMA,
        ],
        compiler_params=pltpu.CompilerParams(
            use_tc_tiling_on_sc=True, needs_layout_passes=False
        ),
    )
    def gather_kernel(ids_hbm, table3_hbm, out_hbm, idx_v, slots_v, rows_v,
                      sem0, sem1):
        wid = lax.axis_index("s") * NC + lax.axis_index("c")
        base = wid * b_per_w
        pltpu.sync_copy(ids_hbm.at[pl.ds(base, b_per_w)], idx_v)

        lanes = lax.iota(jnp.int32, L)

        def fire(m, sem):
            off = (m % 2) * _CH
            tiles = lax.shift_right_logical(idx_v[pl.ds(m * _CH, _CH)], 3)
            for j in range(_CH):
                t = jnp.sum(jnp.where(lanes == j, tiles, 0))
                pltpu.async_copy(
                    table3_hbm.at[pl.ds(t, 1)],
                    slots_v.at[pl.ds(off + j, 1)],
                    sem,
                )

        def drain(sem):
            for _ in range(_CH):
                pltpu.make_async_copy(
                    table3_hbm.at[pl.ds(0, 1)],
                    slots_v.at[pl.ds(0, 1)],
                    sem,
                ).wait()

        def extract(m):
            off = (m % 2) * _CH
            s_vec = idx_v[pl.ds(m * _CH, _CH)] & 7
            slot_vec = off + lanes
            k_vec = m * _CH + lanes
            for j in range(D):
                j_vec = jnp.full((L,), j, jnp.int32)
                row_j = plsc.load_gather(slots_v, [slot_vec, s_vec, j_vec])
                plsc.store_scatter(rows_v, [k_vec, j_vec], row_j)

        def body(m, carry):
            @pl.when(m % 2 == 0)
            def _():
                fire(m, sem0)

            @pl.when(m % 2 == 1)
            def _():
                fire(m, sem1)

            @pl.when(m % 2 == 1)
            def _():
                drain(sem0)
                extract(m - 1)

            @pl.when((m % 2 == 0) & (m > 0))
            def _():
                drain(sem1)
                extract(m - 1)

            return carry

        lax.fori_loop(0, n_chunks, body, 0)
        drain(sem1 if (n_chunks % 2 == 0) else sem0)
        extract(n_chunks - 1)

        pltpu.sync_copy(rows_v, out_hbm.at[pl.ds(base, b_per_w), :])

    return gather_kernel


def kernel(speaker_ids, table):
    ids = speaker_ids.astype(jnp.int32)
    (B,) = ids.shape
    V, D = table.shape
    return _build_gather(B, V, D)(ids, table.reshape(V // 8, 8, D))
